# Initial kernel scaffold; baseline (speedup 1.0000x reference)
import functools

import jax
import jax.numpy as jnp
from jax import lax
from jax.experimental import pallas as pl
from jax.experimental.pallas import tpu as pltpu
from jax.experimental.pallas import tpu_sc as plsc

N = 10000
E = 160000
B = 128
D = 256
HALF = 128
NSUB = 16
NUM_LAYERS = 3

EDGES_PER_SUB = E // NSUB
CHUNK = 128
NFULL = EDGES_PER_SUB // CHUNK
TAIL = EDGES_PER_SUB - NFULL * CHUNK

ROWS_A = 624
ROWS_LAST = N - 15 * ROWS_A
ZROWS = 104

RBLK = 2000
NG = N // RBLK


def _sc_agg_body(src_hbm, dst_hbm, xlo_hbm, xhi_hbm, agglo_hbm, agghi_hbm,
                 sidx0, sidx1, didx0, didx1, rows0, rows1,
                 tsidx, tdidx, trows, zbuf, accum, gsem0, gsem1):
    core = lax.axis_index("c")
    sid = lax.axis_index("s")

    zero16 = jnp.zeros((16,), jnp.float32)

    @pl.loop(0, ZROWS)
    def _(j):
        for k in range(HALF // 16):
            zbuf[j, pl.ds(k * 16, 16)] = zero16

    zbase = sid * ROWS_A
    for j in range(ROWS_A // ZROWS):
        pltpu.sync_copy(zbuf, accum.at[pl.ds(zbase + j * ZROWS, ZROWS)])

    @pl.when(sid == NSUB - 1)
    def _():
        pltpu.sync_copy(zbuf.at[pl.ds(0, 16)],
                        accum.at[pl.ds(16 * ROWS_A, 16)])

    plsc.subcore_barrier()

    ebase = sid * EDGES_PER_SUB

    def _run(x_hbm, agg_hbm):
        def load_idx(c, sb, db):
            pltpu.sync_copy(src_hbm.at[pl.ds(ebase + c * CHUNK, CHUNK)], sb)
            pltpu.sync_copy(dst_hbm.at[pl.ds(ebase + c * CHUNK, CHUNK)], db)

        def start_gather(sb, rb, sem):
            pltpu.async_copy(x_hbm.at[sb], rb, sem)

        def wait_gather(sb, rb, sem):
            pltpu.make_async_copy(x_hbm.at[sb], rb, sem).wait()

        def scatter(rb, db):
            pltpu.sync_copy(rb, accum.at[db], add=True)

        load_idx(0, sidx0, didx0)
        start_gather(sidx0, rows0, gsem0)
        load_idx(1, sidx1, didx1)
        start_gather(sidx1, rows1, gsem1)

        @pl.loop(2, NFULL, step=2)
        def _(c):
            wait_gather(sidx0, rows0, gsem0)
            scatter(rows0, didx0)
            load_idx(c, sidx0, didx0)
            start_gather(sidx0, rows0, gsem0)
            wait_gather(sidx1, rows1, gsem1)
            scatter(rows1, didx1)
            load_idx(c + 1, sidx1, didx1)
            start_gather(sidx1, rows1, gsem1)

        wait_gather(sidx0, rows0, gsem0)
        scatter(rows0, didx0)
        wait_gather(sidx1, rows1, gsem1)
        scatter(rows1, didx1)

        pltpu.sync_copy(src_hbm.at[pl.ds(ebase + NFULL * CHUNK, TAIL)], tsidx)
        pltpu.sync_copy(dst_hbm.at[pl.ds(ebase + NFULL * CHUNK, TAIL)], tdidx)
        pltpu.async_copy(x_hbm.at[tsidx], trows, gsem0).wait()
        pltpu.sync_copy(trows, accum.at[tdidx], add=True)

        plsc.subcore_barrier()

        @pl.when(sid < NSUB - 1)
        def _():
            pltpu.sync_copy(accum.at[pl.ds(sid * ROWS_A, ROWS_A)],
                            agg_hbm.at[pl.ds(sid * ROWS_A, ROWS_A)])

        @pl.when(sid == NSUB - 1)
        def _():
            pltpu.sync_copy(accum.at[pl.ds(15 * ROWS_A, ROWS_LAST)],
                            agg_hbm.at[pl.ds(15 * ROWS_A, ROWS_LAST)])

    @pl.when(core == 0)
    def _():
        _run(xlo_hbm, agglo_hbm)

    @pl.when(core == 1)
    def _():
        _run(xhi_hbm, agghi_hbm)


def _edge_agg_sc(x, src, dst):
    xlo = x[:, :HALF]
    xhi = x[:, HALF:]
    mesh = plsc.VectorSubcoreMesh(core_axis_name="c", subcore_axis_name="s")
    kern = pl.kernel(
        _sc_agg_body,
        out_type=[jax.ShapeDtypeStruct((N, HALF), jnp.float32),
                  jax.ShapeDtypeStruct((N, HALF), jnp.float32)],
        mesh=mesh,
        scratch_types=[
            pltpu.VMEM((CHUNK,), jnp.int32),
            pltpu.VMEM((CHUNK,), jnp.int32),
            pltpu.VMEM((CHUNK,), jnp.int32),
            pltpu.VMEM((CHUNK,), jnp.int32),
            pltpu.VMEM((CHUNK, HALF), jnp.float32),
            pltpu.VMEM((CHUNK, HALF), jnp.float32),
            pltpu.VMEM((TAIL,), jnp.int32),
            pltpu.VMEM((TAIL,), jnp.int32),
            pltpu.VMEM((TAIL, HALF), jnp.float32),
            pltpu.VMEM((ZROWS, HALF), jnp.float32),
            pltpu.VMEM_SHARED((N, HALF), jnp.float32),
            pltpu.SemaphoreType.DMA,
            pltpu.SemaphoreType.DMA,
        ],
    )
    agglo, agghi = kern(src, dst, xlo, xhi)
    return jnp.concatenate([agglo, agghi], axis=1)


_PREC = lax.Precision.HIGHEST


def _dot(a, b):
    return lax.dot(a, b, precision=_PREC, preferred_element_type=jnp.float32)


def _onehot_body(grow_ref, gcol_ref, p_ref, pt_ref):
    grow = grow_ref[0]
    gcol = gcol_ref[0]
    iota_pt = lax.broadcasted_iota(jnp.int32, (B, RBLK), 0)
    pt_ref[...] = (grow == iota_pt).astype(jnp.float32)
    iota_p = lax.broadcasted_iota(jnp.int32, (RBLK, B), 1)
    p_ref[...] = (gcol == iota_p).astype(jnp.float32)


def _build_onehot(gidx):
    grow = gidx.reshape(NG, 1, RBLK)
    gcol = gidx.reshape(NG, RBLK, 1)
    return pl.pallas_call(
        _onehot_body,
        grid=(NG,),
        in_specs=[
            pl.BlockSpec((1, 1, RBLK), lambda i: (i, 0, 0)),
            pl.BlockSpec((1, RBLK, 1), lambda i: (i, 0, 0)),
        ],
        out_specs=[
            pl.BlockSpec((RBLK, B), lambda i: (i, 0)),
            pl.BlockSpec((B, RBLK), lambda i: (0, i)),
        ],
        out_shape=[jax.ShapeDtypeStruct((N, B), jnp.float32),
                   jax.ShapeDtypeStruct((B, N), jnp.float32)],
        compiler_params=pltpu.CompilerParams(
            dimension_semantics=("arbitrary",)),
    )(grow, gcol)


def _layer_body(x_ref, agg_ref, pt_ref, eps1_ref, w1_ref, b1_ref, w2_ref,
                b2_ref, inw_ref, inb_ref, outw_ref, outb_ref,
                xn_ref, gp_ref):
    i = pl.program_id(0)
    h = eps1_ref[0, 0] * x_ref[...] + agg_ref[...]
    h = jax.nn.relu(_dot(h, w1_ref[...]) + b1_ref[...])
    h = _dot(h, w2_ref[...]) + b2_ref[...]
    xn = jax.nn.relu(h)
    xn_ref[...] = xn
    g = jax.nn.relu(_dot(xn, inw_ref[...]) + inb_ref[...])
    acc = _dot(pt_ref[...], g)

    @pl.when(i == 0)
    def _():
        gp_ref[...] = acc

    @pl.when(i > 0)
    def _():
        gp_ref[...] = gp_ref[...] + acc

    @pl.when(i == NG - 1)
    def _():
        gp_ref[...] = jax.nn.relu(
            _dot(gp_ref[...], outw_ref[...]) + outb_ref[...])


def _tc_layer(x, agg, pt, eps1, w1, b1, w2p, b2p, inw, inb, outw, outb):
    def full(r, c):
        return pl.BlockSpec((r, c), lambda i: (0, 0))
    return pl.pallas_call(
        _layer_body,
        grid=(NG,),
        in_specs=[
            pl.BlockSpec((RBLK, D), lambda i: (i, 0)),
            pl.BlockSpec((RBLK, D), lambda i: (i, 0)),
            pl.BlockSpec((B, RBLK), lambda i: (0, i)),
            full(1, 1), full(D, D), full(1, D), full(D, D), full(1, D),
            full(D, D), full(1, D), full(D, D), full(1, D),
        ],
        out_specs=[
            pl.BlockSpec((RBLK, D), lambda i: (i, 0)),
            pl.BlockSpec((B, D), lambda i: (0, 0)),
        ],
        out_shape=[jax.ShapeDtypeStruct((N, D), jnp.float32),
                   jax.ShapeDtypeStruct((B, D), jnp.float32)],
        compiler_params=pltpu.CompilerParams(
            dimension_semantics=("arbitrary",)),
    )(x, agg, pt, eps1, w1, b1, w2p, b2p, inw, inb, outw, outb)


def _reg_body(qn_ref, cn_ref, pq_ref, pc_ref, pqt_ref, pct_ref,
              qg_ref, cg_ref, reg_ref, gi_ref, gj_ref):
    i = pl.program_id(0)
    gdiff = qg_ref[...] - cg_ref[...]
    u = _dot(pq_ref[...], gdiff)
    gi_node = jnp.abs(jnp.sum(qn_ref[...] * u, axis=1, keepdims=True))
    v = _dot(pc_ref[...], gdiff)
    gj_node = jnp.sum(cn_ref[...] * v, axis=1, keepdims=True)
    gi = _dot(pqt_ref[...], gi_node)
    gj = _dot(pct_ref[...], gj_node)

    @pl.when(i == 0)
    def _():
        gi_ref[...] = gi
        gj_ref[...] = gj

    @pl.when(i > 0)
    def _():
        gi_ref[...] = gi_ref[...] + gi
        gj_ref[...] = gj_ref[...] + gj

    @pl.when(i == NG - 1)
    def _():
        a = gi_ref[...]
        b = gj_ref[...]
        reg_ref[...] = (a + b + jnp.abs(a - b)) / float(NUM_LAYERS)


def _tc_reg(qn, cn, pq, pc, pqt, pct, qg, cg):
    return pl.pallas_call(
        _reg_body,
        grid=(NG,),
        in_specs=[
            pl.BlockSpec((RBLK, D), lambda i: (i, 0)),
            pl.BlockSpec((RBLK, D), lambda i: (i, 0)),
            pl.BlockSpec((RBLK, B), lambda i: (i, 0)),
            pl.BlockSpec((RBLK, B), lambda i: (i, 0)),
            pl.BlockSpec((B, RBLK), lambda i: (0, i)),
            pl.BlockSpec((B, RBLK), lambda i: (0, i)),
            pl.BlockSpec((B, D), lambda i: (0, 0)),
            pl.BlockSpec((B, D), lambda i: (0, 0)),
        ],
        out_specs=pl.BlockSpec((B, 1), lambda i: (0, 0)),
        out_shape=jax.ShapeDtypeStruct((B, 1), jnp.float32),
        scratch_shapes=[pltpu.VMEM((B, 1), jnp.float32),
                        pltpu.VMEM((B, 1), jnp.float32)],
        compiler_params=pltpu.CompilerParams(
            dimension_semantics=("arbitrary",)),
    )(qn, cn, pq, pc, pqt, pct, qg, cg)


T = 16


def _scores_body(qg0_ref, qg1_ref, qg2_ref, cg0_ref, cg1_ref, cg2_ref,
                 tnw_ref, tnwb_ref, tnb_ref, ts1w_ref, ts1b_ref,
                 ts2w_ref, ts2b_ref,
                 mw1_ref, mb1_ref, mw2_ref, mb2_ref, ms1w_ref, ms1b_ref,
                 ms2w_ref, ms2b_ref, ab_ref, out_ref):
    q = qg2_ref[...]
    c = cg2_ref[...]
    tmp = _dot(q, tnw_ref[...])
    cols = []
    for t in range(T):
        cols.append(jnp.sum(tmp[:, t * D:(t + 1) * D] * c,
                            axis=1, keepdims=True))
    scoring = jnp.concatenate(cols, axis=1)
    comb = jnp.concatenate([q, c], axis=1)
    block = _dot(comb, tnwb_ref[...])
    s = jax.nn.relu(scoring + block + tnb_ref[...])
    s = jax.nn.relu(_dot(s, ts1w_ref[...]) + ts1b_ref[...])
    s1 = _dot(s, ts2w_ref[...]) + ts2b_ref[...]

    qcat = jnp.concatenate([qg0_ref[...], qg1_ref[...], qg2_ref[...]], axis=1)
    ccat = jnp.concatenate([cg0_ref[...], cg1_ref[...], cg2_ref[...]], axis=1)
    diff = jnp.exp(-jnp.square(qcat - ccat))
    h = jax.nn.relu(_dot(diff, mw1_ref[...]) + mb1_ref[...])
    h = jnp.tanh(_dot(h, mw2_ref[...]) + mb2_ref[...])
    s = jax.nn.relu(_dot(h, ms1w_ref[...]) + ms1b_ref[...])
    s2 = _dot(s, ms2w_ref[...]) + ms2b_ref[...]

    out_ref[...] = ab_ref[0, 0] * s1 + ab_ref[0, 1] * s2


def _tc_scores(qg, cg, p):
    tnw = p['tn_W'].transpose(0, 2, 1).reshape(D, T * D)
    tnwb = p['tn_Wblock'].T
    tnb = p['tn_bias'].reshape(1, T)
    ab = jnp.stack([p['alpha'][0], p['beta'][0]]).reshape(1, 2)
    args = [qg[0], qg[1], qg[2], cg[0], cg[1], cg[2],
            tnw, tnwb, tnb,
            p['tn_s1_W'], p['tn_s1_b'].reshape(1, T),
            p['tn_s2_W'], p['tn_s2_b'].reshape(1, 1),
            p['mk_W1'], p['mk_b1'].reshape(1, -1),
            p['mk_W2'], p['mk_b2'].reshape(1, -1),
            p['mk_s1_W'], p['mk_s1_b'].reshape(1, -1),
            p['mk_s2_W'], p['mk_s2_b'].reshape(1, 1),
            ab]
    return pl.pallas_call(
        _scores_body,
        out_shape=jax.ShapeDtypeStruct((B, 1), jnp.float32),
    )(*args)


def _encode(x, src, dst, pt, p):
    graph_feats = []
    for i in range(NUM_LAYERS):
        agg = _edge_agg_sc(x, src, dst)
        eps1 = (1.0 + p[f'gin_eps_{i}']).reshape(1, 1)
        gamma = p[f'bn_gamma_{i}']
        w2p = p[f'gin_W2_{i}'] * gamma[None, :]
        b2p = (p[f'gin_b2_{i}'] * gamma + p[f'bn_beta_{i}']).reshape(1, D)
        x, g = _tc_layer(
            x, agg, pt, eps1,
            p[f'gin_W1_{i}'], p[f'gin_b1_{i}'].reshape(1, D),
            w2p, b2p,
            p[f'in_W_{i}'], p[f'in_b_{i}'].reshape(1, D),
            p[f'out_W_{i}'], p[f'out_b_{i}'].reshape(1, D))
        graph_feats.append(g)
    return graph_feats, x


def kernel(query_x, corpus_x, params, query_edge_index, query_graph_idx,
           corpus_edge_index, corpus_graph_idx, batch_size):
    p = params
    pq, pqt = _build_onehot(query_graph_idx)
    pc, pct = _build_onehot(corpus_graph_idx)
    qg, qn = _encode(query_x, query_edge_index[0], query_edge_index[1],
                     pqt, p)
    cg, cn = _encode(corpus_x, corpus_edge_index[0], corpus_edge_index[1],
                     pct, p)
    score = _tc_scores(qg, cg, p).reshape(-1)
    reg = _tc_reg(qn, cn, pq, pc, pqt, pct, qg[-1], cg[-1]).reshape(-1)
    return (score, reg)


# speedup vs baseline: 4.6512x; 4.6512x over previous
import functools

import jax
import jax.numpy as jnp
from jax import lax
from jax.experimental import pallas as pl
from jax.experimental.pallas import tpu as pltpu
from jax.experimental.pallas import tpu_sc as plsc

N = 10000
E = 160000
B = 128
D = 256
HALF = 128
NSUB = 16
NUM_LAYERS = 3

EDGES_PER_SUB = E // NSUB
CHUNK = 128
NFULL = EDGES_PER_SUB // CHUNK
TAIL = EDGES_PER_SUB - NFULL * CHUNK

ROWS_A = 624
ROWS_LAST = N - 15 * ROWS_A
ZROWS = 104

RBLK = 2000
NG = N // RBLK


def _sc_agg_body(src_hbm, dst_hbm, xlo_hbm, xhi_hbm, agglo_hbm, agghi_hbm,
                 sidx0, sidx1, didx0, didx1, rows0, rows1,
                 tsidx, tdidx, trows, zbuf, accum, gsem0, gsem1):
    core = lax.axis_index("c")
    sid = lax.axis_index("s")

    zero16 = jnp.zeros((16,), jnp.float32)

    @pl.loop(0, ZROWS)
    def _(j):
        for k in range(HALF // 16):
            zbuf[j, pl.ds(k * 16, 16)] = zero16

    zbase = sid * ROWS_A
    for j in range(ROWS_A // ZROWS):
        pltpu.sync_copy(zbuf, accum.at[pl.ds(zbase + j * ZROWS, ZROWS)])

    @pl.when(sid == NSUB - 1)
    def _():
        pltpu.sync_copy(zbuf.at[pl.ds(0, 16)],
                        accum.at[pl.ds(16 * ROWS_A, 16)])

    plsc.subcore_barrier()

    ebase = sid * EDGES_PER_SUB

    def _run(x_hbm, agg_hbm):
        def load_idx(c, sb, db):
            pltpu.sync_copy(src_hbm.at[pl.ds(ebase + c * CHUNK, CHUNK)], sb)
            pltpu.sync_copy(dst_hbm.at[pl.ds(ebase + c * CHUNK, CHUNK)], db)

        def start_gather(sb, rb, sem):
            pltpu.async_copy(x_hbm.at[sb], rb, sem)

        def wait_gather(sb, rb, sem):
            pltpu.make_async_copy(x_hbm.at[sb], rb, sem).wait()

        def scatter(rb, db):
            pltpu.sync_copy(rb, accum.at[db], add=True)

        load_idx(0, sidx0, didx0)
        start_gather(sidx0, rows0, gsem0)
        load_idx(1, sidx1, didx1)
        start_gather(sidx1, rows1, gsem1)

        @pl.loop(2, NFULL, step=2)
        def _(c):
            wait_gather(sidx0, rows0, gsem0)
            scatter(rows0, didx0)
            load_idx(c, sidx0, didx0)
            start_gather(sidx0, rows0, gsem0)
            wait_gather(sidx1, rows1, gsem1)
            scatter(rows1, didx1)
            load_idx(c + 1, sidx1, didx1)
            start_gather(sidx1, rows1, gsem1)

        wait_gather(sidx0, rows0, gsem0)
        scatter(rows0, didx0)
        wait_gather(sidx1, rows1, gsem1)
        scatter(rows1, didx1)

        pltpu.sync_copy(src_hbm.at[pl.ds(ebase + NFULL * CHUNK, TAIL)], tsidx)
        pltpu.sync_copy(dst_hbm.at[pl.ds(ebase + NFULL * CHUNK, TAIL)], tdidx)
        pltpu.async_copy(x_hbm.at[tsidx], trows, gsem0).wait()
        pltpu.sync_copy(trows, accum.at[tdidx], add=True)

        plsc.subcore_barrier()

        @pl.when(sid < NSUB - 1)
        def _():
            pltpu.sync_copy(accum.at[pl.ds(sid * ROWS_A, ROWS_A)],
                            agg_hbm.at[pl.ds(sid * ROWS_A, ROWS_A)])

        @pl.when(sid == NSUB - 1)
        def _():
            pltpu.sync_copy(accum.at[pl.ds(15 * ROWS_A, ROWS_LAST)],
                            agg_hbm.at[pl.ds(15 * ROWS_A, ROWS_LAST)])

    @pl.when(core == 0)
    def _():
        _run(xlo_hbm, agglo_hbm)

    @pl.when(core == 1)
    def _():
        _run(xhi_hbm, agghi_hbm)


def _edge_agg_sc(x, src, dst):
    xlo = x[:, :HALF]
    xhi = x[:, HALF:]
    mesh = plsc.VectorSubcoreMesh(core_axis_name="c", subcore_axis_name="s")
    kern = pl.kernel(
        _sc_agg_body,
        out_type=[jax.ShapeDtypeStruct((N, HALF), jnp.float32),
                  jax.ShapeDtypeStruct((N, HALF), jnp.float32)],
        mesh=mesh,
        scratch_types=[
            pltpu.VMEM((CHUNK,), jnp.int32),
            pltpu.VMEM((CHUNK,), jnp.int32),
            pltpu.VMEM((CHUNK,), jnp.int32),
            pltpu.VMEM((CHUNK,), jnp.int32),
            pltpu.VMEM((CHUNK, HALF), jnp.float32),
            pltpu.VMEM((CHUNK, HALF), jnp.float32),
            pltpu.VMEM((TAIL,), jnp.int32),
            pltpu.VMEM((TAIL,), jnp.int32),
            pltpu.VMEM((TAIL, HALF), jnp.float32),
            pltpu.VMEM((ZROWS, HALF), jnp.float32),
            pltpu.VMEM_SHARED((N, HALF), jnp.float32),
            pltpu.SemaphoreType.DMA,
            pltpu.SemaphoreType.DMA,
        ],
    )
    agglo, agghi = kern(src, dst, xlo, xhi)
    return jnp.concatenate([agglo, agghi], axis=1)


_PREC = None


def _dot(a, b):
    return lax.dot(a, b, precision=_PREC, preferred_element_type=jnp.float32)


def _onehot_body(grow_ref, gcol_ref, p_ref, pt_ref):
    grow = grow_ref[0]
    gcol = gcol_ref[0]
    iota_pt = lax.broadcasted_iota(jnp.int32, (B, RBLK), 0)
    pt_ref[0] = (grow == iota_pt).astype(jnp.float32)
    iota_p = lax.broadcasted_iota(jnp.int32, (RBLK, B), 1)
    p_ref[...] = (gcol == iota_p).astype(jnp.float32)


def _build_onehot(gidx):
    grow = gidx.reshape(NG, 1, RBLK)
    gcol = gidx.reshape(NG, RBLK, 1)
    return pl.pallas_call(
        _onehot_body,
        grid=(NG,),
        in_specs=[
            pl.BlockSpec((1, 1, RBLK), lambda i: (i, 0, 0)),
            pl.BlockSpec((1, RBLK, 1), lambda i: (i, 0, 0)),
        ],
        out_specs=[
            pl.BlockSpec((RBLK, B), lambda i: (i, 0)),
            pl.BlockSpec((1, B, RBLK), lambda i: (i, 0, 0)),
        ],
        out_shape=[jax.ShapeDtypeStruct((N, B), jnp.float32),
                   jax.ShapeDtypeStruct((NG, B, RBLK), jnp.float32)],
        compiler_params=pltpu.CompilerParams(
            dimension_semantics=("arbitrary",)),
    )(grow, gcol)


def _layer_body(x_ref, agg_ref, pt_ref, eps1_ref, w1_ref, b1_ref, w2_ref,
                b2_ref, inw_ref, inb_ref, outw_ref, outb_ref,
                xn_ref, gp_ref):
    i = pl.program_id(0)
    h = eps1_ref[0, 0] * x_ref[...] + agg_ref[...]
    pt = pt_ref[0]
    h = jax.nn.relu(_dot(h, w1_ref[...]) + b1_ref[...])
    h = _dot(h, w2_ref[...]) + b2_ref[...]
    xn = jax.nn.relu(h)
    xn_ref[...] = xn
    g = jax.nn.relu(_dot(xn, inw_ref[...]) + inb_ref[...])
    acc = _dot(pt, g)

    @pl.when(i == 0)
    def _():
        gp_ref[...] = acc

    @pl.when(i > 0)
    def _():
        gp_ref[...] = gp_ref[...] + acc

    @pl.when(i == NG - 1)
    def _():
        gp_ref[...] = jax.nn.relu(
            _dot(gp_ref[...], outw_ref[...]) + outb_ref[...])


def _tc_layer(x, agg, pt, eps1, w1, b1, w2p, b2p, inw, inb, outw, outb):
    def full(r, c):
        return pl.BlockSpec((r, c), lambda i: (0, 0))
    return pl.pallas_call(
        _layer_body,
        grid=(NG,),
        in_specs=[
            pl.BlockSpec((RBLK, D), lambda i: (i, 0)),
            pl.BlockSpec((RBLK, D), lambda i: (i, 0)),
            pl.BlockSpec((1, B, RBLK), lambda i: (i, 0, 0)),
            full(1, 1), full(D, D), full(1, D), full(D, D), full(1, D),
            full(D, D), full(1, D), full(D, D), full(1, D),
        ],
        out_specs=[
            pl.BlockSpec((RBLK, D), lambda i: (i, 0)),
            pl.BlockSpec((B, D), lambda i: (0, 0)),
        ],
        out_shape=[jax.ShapeDtypeStruct((N, D), jnp.float32),
                   jax.ShapeDtypeStruct((B, D), jnp.float32)],
        compiler_params=pltpu.CompilerParams(
            dimension_semantics=("arbitrary",)),
    )(x, agg, pt, eps1, w1, b1, w2p, b2p, inw, inb, outw, outb)


def _reg_body(qn_ref, cn_ref, pq_ref, pc_ref, pqt_ref, pct_ref,
              qg_ref, cg_ref, reg_ref, gi_ref, gj_ref):
    i = pl.program_id(0)
    gdiff = qg_ref[...] - cg_ref[...]
    u = _dot(pq_ref[...], gdiff)
    gi_node = jnp.abs(jnp.sum(qn_ref[...] * u, axis=1, keepdims=True))
    v = _dot(pc_ref[...], gdiff)
    gj_node = jnp.sum(cn_ref[...] * v, axis=1, keepdims=True)
    gi = _dot(pqt_ref[0], gi_node)
    gj = _dot(pct_ref[0], gj_node)

    @pl.when(i == 0)
    def _():
        gi_ref[...] = gi
        gj_ref[...] = gj

    @pl.when(i > 0)
    def _():
        gi_ref[...] = gi_ref[...] + gi
        gj_ref[...] = gj_ref[...] + gj

    @pl.when(i == NG - 1)
    def _():
        a = gi_ref[...]
        b = gj_ref[...]
        reg_ref[...] = (a + b + jnp.abs(a - b)) / float(NUM_LAYERS)


def _tc_reg(qn, cn, pq, pc, pqt, pct, qg, cg):
    return pl.pallas_call(
        _reg_body,
        grid=(NG,),
        in_specs=[
            pl.BlockSpec((RBLK, D), lambda i: (i, 0)),
            pl.BlockSpec((RBLK, D), lambda i: (i, 0)),
            pl.BlockSpec((RBLK, B), lambda i: (i, 0)),
            pl.BlockSpec((RBLK, B), lambda i: (i, 0)),
            pl.BlockSpec((1, B, RBLK), lambda i: (i, 0, 0)),
            pl.BlockSpec((1, B, RBLK), lambda i: (i, 0, 0)),
            pl.BlockSpec((B, D), lambda i: (0, 0)),
            pl.BlockSpec((B, D), lambda i: (0, 0)),
        ],
        out_specs=pl.BlockSpec((B, 1), lambda i: (0, 0)),
        out_shape=jax.ShapeDtypeStruct((B, 1), jnp.float32),
        scratch_shapes=[pltpu.VMEM((B, 1), jnp.float32),
                        pltpu.VMEM((B, 1), jnp.float32)],
        compiler_params=pltpu.CompilerParams(
            dimension_semantics=("arbitrary",)),
    )(qn, cn, pq, pc, pqt, pct, qg, cg)


T = 16


def _scores_body(qg0_ref, qg1_ref, qg2_ref, cg0_ref, cg1_ref, cg2_ref,
                 tnw_ref, tnwb_ref, tnb_ref, ts1w_ref, ts1b_ref,
                 ts2w_ref, ts2b_ref,
                 mw1_ref, mb1_ref, mw2_ref, mb2_ref, ms1w_ref, ms1b_ref,
                 ms2w_ref, ms2b_ref, ab_ref, out_ref):
    q = qg2_ref[...]
    c = cg2_ref[...]
    tmp = _dot(q, tnw_ref[...])
    cols = []
    for t in range(T):
        cols.append(jnp.sum(tmp[:, t * D:(t + 1) * D] * c,
                            axis=1, keepdims=True))
    scoring = jnp.concatenate(cols, axis=1)
    comb = jnp.concatenate([q, c], axis=1)
    block = _dot(comb, tnwb_ref[...])
    s = jax.nn.relu(scoring + block + tnb_ref[...])
    s = jax.nn.relu(_dot(s, ts1w_ref[...]) + ts1b_ref[...])
    s1 = _dot(s, ts2w_ref[...]) + ts2b_ref[...]

    qcat = jnp.concatenate([qg0_ref[...], qg1_ref[...], qg2_ref[...]], axis=1)
    ccat = jnp.concatenate([cg0_ref[...], cg1_ref[...], cg2_ref[...]], axis=1)
    diff = jnp.exp(-jnp.square(qcat - ccat))
    h = jax.nn.relu(_dot(diff, mw1_ref[...]) + mb1_ref[...])
    h = jnp.tanh(_dot(h, mw2_ref[...]) + mb2_ref[...])
    s = jax.nn.relu(_dot(h, ms1w_ref[...]) + ms1b_ref[...])
    s2 = _dot(s, ms2w_ref[...]) + ms2b_ref[...]

    out_ref[...] = ab_ref[0, 0] * s1 + ab_ref[0, 1] * s2


def _tc_scores(qg, cg, p):
    tnw = p['tn_W'].transpose(0, 2, 1).reshape(D, T * D)
    tnwb = p['tn_Wblock'].T
    tnb = p['tn_bias'].reshape(1, T)
    ab = jnp.stack([p['alpha'][0], p['beta'][0]]).reshape(1, 2)
    args = [qg[0], qg[1], qg[2], cg[0], cg[1], cg[2],
            tnw, tnwb, tnb,
            p['tn_s1_W'], p['tn_s1_b'].reshape(1, T),
            p['tn_s2_W'], p['tn_s2_b'].reshape(1, 1),
            p['mk_W1'], p['mk_b1'].reshape(1, -1),
            p['mk_W2'], p['mk_b2'].reshape(1, -1),
            p['mk_s1_W'], p['mk_s1_b'].reshape(1, -1),
            p['mk_s2_W'], p['mk_s2_b'].reshape(1, 1),
            ab]
    return pl.pallas_call(
        _scores_body,
        out_shape=jax.ShapeDtypeStruct((B, 1), jnp.float32),
    )(*args)


def _encode(x, src, dst, pt, p):
    graph_feats = []
    for i in range(NUM_LAYERS):
        agg = _edge_agg_sc(x, src, dst)
        eps1 = (1.0 + p[f'gin_eps_{i}']).reshape(1, 1)
        gamma = p[f'bn_gamma_{i}']
        w2p = p[f'gin_W2_{i}'] * gamma[None, :]
        b2p = (p[f'gin_b2_{i}'] * gamma + p[f'bn_beta_{i}']).reshape(1, D)
        x, g = _tc_layer(
            x, agg, pt, eps1,
            p[f'gin_W1_{i}'], p[f'gin_b1_{i}'].reshape(1, D),
            w2p, b2p,
            p[f'in_W_{i}'], p[f'in_b_{i}'].reshape(1, D),
            p[f'out_W_{i}'], p[f'out_b_{i}'].reshape(1, D))
        graph_feats.append(g)
    return graph_feats, x


def kernel(query_x, corpus_x, params, query_edge_index, query_graph_idx,
           corpus_edge_index, corpus_graph_idx, batch_size):
    p = params
    pq, pqt = _build_onehot(query_graph_idx)
    pc, pct = _build_onehot(corpus_graph_idx)
    qg, qn = _encode(query_x, query_edge_index[0], query_edge_index[1],
                     pqt, p)
    cg, cn = _encode(corpus_x, corpus_edge_index[0], corpus_edge_index[1],
                     pct, p)
    score = _tc_scores(qg, cg, p).reshape(-1)
    reg = _tc_reg(qn, cn, pq, pc, pqt, pct, qg[-1], cg[-1]).reshape(-1)
    return (score, reg)
